# Initial kernel scaffold; baseline (speedup 1.0000x reference)
import jax
import jax.numpy as jnp
from jax.experimental import pallas as pl


def kernel(mp_idx, mp_val, qx, kx):
    raise NotImplementedError("write your pallas kernel here")


# speedup vs baseline: 9.1881x; 9.1881x over previous
import functools

import jax
import jax.numpy as jnp
from jax import lax
from jax.experimental import pallas as pl
from jax.experimental.pallas import tpu as pltpu
from jax.experimental.pallas import tpu_sc as plsc

B = 16384
T = 1284
W = 2576
NC = 2
NS = 16
NW = NC * NS
BPW = B // NW
L = 16
K = 32


def _annual_table_tc(qx, kx):
    def body(qx_ref, kx_ref, o_ref):
        o_ref[...] = (jnp.power(qx_ref[...] + 1.0, 1.0 / 12.0) - 1.0) * (
            1.0 - kx_ref[...]
        )

    return pl.pallas_call(
        body,
        out_shape=jax.ShapeDtypeStruct(qx.shape, jnp.float32),
    )(qx, kx)


def _make_sc_kernel():
    mesh = plsc.VectorSubcoreMesh(core_axis_name="c", subcore_axis_name="s")

    @functools.partial(
        pl.kernel,
        out_type=jax.ShapeDtypeStruct((B, T), jnp.float32),
        mesh=mesh,
        compiler_params=pltpu.CompilerParams(use_tc_tiling_on_sc=False),
        scratch_types=[
            pltpu.VMEM((16 * W,), jnp.float32),
            pltpu.VMEM((BPW,), jnp.int32),
            pltpu.VMEM((BPW,), jnp.int32),
            pltpu.VMEM((BPW,), jnp.int32),
            pltpu.VMEM((BPW,), jnp.int32),
            pltpu.SemaphoreType.DMA,
        ],
    )
    def sc_kern(
        t8_hbm, sex_hbm, age_hbm, dur_hbm, out_hbm,
        table_v, sex_v, age_v, dur_v, start_v, sem_out,
    ):
        wid = lax.axis_index("s") * NC + lax.axis_index("c")
        base = wid * BPW

        pltpu.sync_copy(sex_hbm.at[pl.ds(base, BPW)], sex_v)
        pltpu.sync_copy(age_hbm.at[pl.ds(base, BPW)], age_v)
        pltpu.sync_copy(dur_hbm.at[pl.ds(base, BPW)], dur_v)
        pltpu.sync_copy(t8_hbm, table_v)

        for g in range(BPW // L):
            sl = pl.ds(g * L, L)
            sex = sex_v[sl]
            age = age_v[sl]
            dur = dur_v[sl]
            mth = age * 12 + dur
            p = jnp.bitwise_and(mth, 7)
            start = (p * 2 + sex) * W + (mth - p)
            start_v[pl.ds(g * L, L)] = start

        def fire(g, carry):
            sv = start_v[pl.ds(g * L, L)]
            row0 = base + g * L
            for i in range(L):
                s = pl.multiple_of(sv[i], 8)
                pltpu.sync_copy(table_v.at[pl.ds(s, T)], out_hbm.at[row0 + i])
            return carry

        lax.fori_loop(0, BPW // L, fire, 0)

    return sc_kern


_SC_KERN = _make_sc_kernel()


def kernel(mp_idx, mp_val, qx, kx):
    q_ann = _annual_table_tc(qx, kx)
    q_mth = jnp.repeat(q_ann, 12, axis=1)
    t_pad = jnp.zeros((2, W + 8), jnp.float32).at[:, :T].set(q_mth)
    t8 = jnp.stack([t_pad[:, p : p + W] for p in range(8)])
    t8_flat = t8.reshape(16 * W)
    return _SC_KERN(t8_flat, mp_idx[:, 0], mp_idx[:, 1], mp_idx[:, 4])


# speedup vs baseline: 9.8017x; 1.0668x over previous
import functools

import jax
import jax.numpy as jnp
from jax import lax
from jax.experimental import pallas as pl
from jax.experimental.pallas import tpu as pltpu
from jax.experimental.pallas import tpu_sc as plsc

B = 16384
T = 1284
W = 2576
NC = 2
NS = 16
NW = NC * NS
BPW = B // NW
L = 16
K = 32


def _annual_table_tc(qx, kx):
    def body(qx_ref, kx_ref, o_ref):
        o_ref[...] = (jnp.power(qx_ref[...] + 1.0, 1.0 / 12.0) - 1.0) * (
            1.0 - kx_ref[...]
        )

    return pl.pallas_call(
        body,
        out_shape=jax.ShapeDtypeStruct(qx.shape, jnp.float32),
    )(qx, kx)


def _make_sc_kernel():
    mesh = plsc.VectorSubcoreMesh(core_axis_name="c", subcore_axis_name="s")

    @functools.partial(
        pl.kernel,
        out_type=jax.ShapeDtypeStruct((B, T), jnp.float32),
        mesh=mesh,
        compiler_params=pltpu.CompilerParams(use_tc_tiling_on_sc=False),
        scratch_types=[
            pltpu.VMEM((16 * W,), jnp.float32),
            pltpu.VMEM((BPW,), jnp.int32),
            pltpu.VMEM((BPW,), jnp.int32),
            pltpu.VMEM((BPW,), jnp.int32),
            pltpu.VMEM((BPW,), jnp.int32),
            pltpu.SemaphoreType.DMA,
        ],
    )
    def sc_kern(
        t8_hbm, sex_hbm, age_hbm, dur_hbm, out_hbm,
        table_v, sex_v, age_v, dur_v, start_v, sem_out,
    ):
        wid = lax.axis_index("s") * NC + lax.axis_index("c")
        base = wid * BPW

        pltpu.sync_copy(sex_hbm.at[pl.ds(base, BPW)], sex_v)
        pltpu.sync_copy(age_hbm.at[pl.ds(base, BPW)], age_v)
        pltpu.sync_copy(dur_hbm.at[pl.ds(base, BPW)], dur_v)
        pltpu.sync_copy(t8_hbm, table_v)

        for g in range(BPW // L):
            sl = pl.ds(g * L, L)
            sex = sex_v[sl]
            age = age_v[sl]
            dur = dur_v[sl]
            mth = age * 12 + dur
            p = jnp.bitwise_and(mth, 7)
            start = (p * 2 + sex) * W + (mth - p)
            start_v[pl.ds(g * L, L)] = start

        def wait_one_row():
            pltpu.make_async_copy(
                table_v.at[pl.ds(0, T)], out_hbm.at[base], sem_out
            ).wait()

        def fire(g, carry):
            sv = start_v[pl.ds(g * L, L)]
            row0 = base + g * L
            for i in range(L):
                s = pl.multiple_of(sv[i], 8)
                pltpu.make_async_copy(
                    table_v.at[pl.ds(s, T)], out_hbm.at[row0 + i], sem_out
                ).start()

            @pl.when(g >= 1)
            def _():
                for _i in range(L):
                    wait_one_row()

            return carry

        lax.fori_loop(0, BPW // L, fire, 0)
        for _i in range(L):
            wait_one_row()

    return sc_kern


_SC_KERN = _make_sc_kernel()


def kernel(mp_idx, mp_val, qx, kx):
    q_ann = _annual_table_tc(qx, kx)
    q_mth = jnp.repeat(q_ann, 12, axis=1)
    t_pad = jnp.zeros((2, W + 8), jnp.float32).at[:, :T].set(q_mth)
    t8 = jnp.stack([t_pad[:, p : p + W] for p in range(8)])
    t8_flat = t8.reshape(16 * W)
    return _SC_KERN(t8_flat, mp_idx[:, 0], mp_idx[:, 1], mp_idx[:, 4])
